# Initial kernel scaffold; baseline (speedup 1.0000x reference)
import jax
import jax.numpy as jnp
from jax.experimental import pallas as pl


def kernel(x, xyz):
    raise NotImplementedError("write your pallas kernel here")


# speedup vs baseline: 12.2266x; 12.2266x over previous
import functools

import jax
import jax.numpy as jnp
from jax import lax
from jax.experimental import pallas as pl
from jax.experimental.pallas import tpu as pltpu

_B, _D, _N = 4, 128, 8192
_M = 2048
_K = 4
_NR, _NC = 64, 128


def _sum2(v):
    return jnp.sum(jnp.sum(v, axis=1, keepdims=True), axis=0, keepdims=True)


def _max2(v):
    return jnp.max(jnp.max(v, axis=1, keepdims=True), axis=0, keepdims=True)


def _min2(v):
    return jnp.min(jnp.min(v, axis=1, keepdims=True), axis=0, keepdims=True)


def _fps_body(xyz_ref, ind_ref, cxyz_ref, dist_ref):
    iota2 = (lax.broadcasted_iota(jnp.int32, (_NR, _NC), 0) * _NC
             + lax.broadcasted_iota(jnp.int32, (_NR, _NC), 1))
    for b in range(_B):
        dist_ref[b] = jnp.full((_NR, _NC), jnp.inf, jnp.float32)

    def body(i, far):
        ind_ref[pl.ds(i, 1), :] = far
        cents = []
        new_far = []
        for b in range(_B):
            fb = far[0:1, b:b + 1]
            onehot = iota2 == fb
            xb = xyz_ref[b, 0]
            yb = xyz_ref[b, 1]
            zb = xyz_ref[b, 2]
            cx = _sum2(jnp.where(onehot, xb, 0.0))
            cy = _sum2(jnp.where(onehot, yb, 0.0))
            cz = _sum2(jnp.where(onehot, zb, 0.0))
            cents += [cx, cy, cz]
            dx = xb - cx
            dy = yb - cy
            dz = zb - cz
            d2 = (dx * dx + dy * dy) + dz * dz
            nd = jnp.minimum(dist_ref[b], d2)
            dist_ref[b] = nd
            mx = _max2(nd)
            idx = _min2(jnp.where(nd == mx, iota2, _N))
            new_far.append(idx)
        cxyz_ref[pl.ds(i, 1), :] = jnp.concatenate(cents, axis=1)
        return jnp.concatenate(new_far, axis=1)

    lax.fori_loop(0, _M, body, jnp.zeros((1, _B), jnp.int32))


@jax.jit
def _fps(xyz4):
    return pl.pallas_call(
        _fps_body,
        out_shape=[
            jax.ShapeDtypeStruct((_M, _B), jnp.int32),
            jax.ShapeDtypeStruct((_M, 3 * _B), jnp.float32),
        ],
        scratch_shapes=[pltpu.VMEM((_B, _NR, _NC), jnp.float32)],
    )(xyz4)


_MT = 256


def _top4_body(sq_ref, xyz_ref, out_ref):
    s = sq_ref[0]
    s0 = s[:, 0:1]
    s1 = s[:, 1:2]
    s2 = s[:, 2:3]
    x0 = xyz_ref[0, 0:1, :]
    x1 = xyz_ref[0, 1:2, :]
    x2 = xyz_ref[0, 2:3, :]
    a2 = (s0 * s0 + s1 * s1) + s2 * s2
    b2 = (x0 * x0 + x1 * x1) + x2 * x2

    def _bf(v):
        return v.astype(jnp.bfloat16).astype(jnp.float32)

    ab = (_bf(s0) * _bf(x0) + _bf(s1) * _bf(x1)) + _bf(s2) * _bf(x2)
    d2 = jnp.maximum((a2 + b2) - 2.0 * ab, 0.0)
    work = jnp.sqrt(d2)
    iota = lax.broadcasted_iota(jnp.int32, (_MT, _N), 1)
    cols = []
    for k in range(_K):
        mn = jnp.min(work, axis=1, keepdims=True)
        ik = jnp.min(jnp.where(work == mn, iota, _N), axis=1, keepdims=True)
        cols.append(ik)
        if k < _K - 1:
            work = jnp.where(iota == ik, jnp.inf, work)
    out_ref[0] = jnp.concatenate(cols, axis=1)


@jax.jit
def _top4(sq0, xyz):
    return pl.pallas_call(
        _top4_body,
        grid=(_B, _M // _MT),
        in_specs=[
            pl.BlockSpec((1, _MT, 3), lambda b, m: (b, m, 0)),
            pl.BlockSpec((1, 3, _N), lambda b, m: (b, 0, 0)),
        ],
        out_specs=pl.BlockSpec((1, _MT, _K), lambda b, m: (b, m, 0)),
        out_shape=jax.ShapeDtypeStruct((_B, _M, _K), jnp.int32),
    )(sq0, xyz)


def kernel(x, xyz):
    xyz4 = xyz.reshape(_B, 3, _NR, _NC)
    ind_t, cxyz = _fps(xyz4)
    sample_ind = ind_t.T
    sq0 = cxyz.reshape(_M, _B, 3).transpose(1, 0, 2)
    neighbor_ind = _top4(sq0, xyz)

    ind_flat = neighbor_ind.reshape(_B, 1, _M * _K)
    gx = jnp.take_along_axis(
        xyz, jnp.broadcast_to(ind_flat, (_B, 3, _M * _K)), axis=2
    ).reshape(_B, 3, _M, _K)
    sample_xyz = gx.mean(axis=-1)
    gf = jnp.take_along_axis(
        x, jnp.broadcast_to(ind_flat, (_B, _D, _M * _K)), axis=2
    ).reshape(_B, _D, _M, _K)
    sample_x = gf.max(axis=-1)
    return (sample_x, sample_xyz, sample_ind, neighbor_ind)


# speedup vs baseline: 18.7385x; 1.5326x over previous
import functools

import jax
import jax.numpy as jnp
from jax import lax
from jax.experimental import pallas as pl
from jax.experimental.pallas import tpu as pltpu

_B, _D, _N = 4, 128, 8192
_M = 2048
_K = 4
_NR, _NC = 64, 128


def _sum2(v):
    return jnp.sum(jnp.sum(v, axis=1, keepdims=True), axis=0, keepdims=True)


def _max2(v):
    return jnp.max(jnp.max(v, axis=1, keepdims=True), axis=0, keepdims=True)


def _min2(v):
    return jnp.min(jnp.min(v, axis=1, keepdims=True), axis=0, keepdims=True)


def _fps_body(xyz_ref, ind_ref, cxyz_ref, dist_ref):
    iota2 = (lax.broadcasted_iota(jnp.int32, (_NR, _NC), 0) * _NC
             + lax.broadcasted_iota(jnp.int32, (_NR, _NC), 1))
    for b in range(_B):
        dist_ref[b] = jnp.full((_NR, _NC), jnp.inf, jnp.float32)

    def body(i, far):
        ind_ref[pl.ds(i, 1), :] = far
        cents = []
        new_far = []
        for b in range(_B):
            fb = far[0:1, b:b + 1]
            onehot = iota2 == fb
            xb = xyz_ref[b, 0]
            yb = xyz_ref[b, 1]
            zb = xyz_ref[b, 2]
            cx = _sum2(jnp.where(onehot, xb, 0.0))
            cy = _sum2(jnp.where(onehot, yb, 0.0))
            cz = _sum2(jnp.where(onehot, zb, 0.0))
            cents += [cx, cy, cz]
            dx = xb - cx
            dy = yb - cy
            dz = zb - cz
            d2 = (dx * dx + dy * dy) + dz * dz
            nd = jnp.minimum(dist_ref[b], d2)
            dist_ref[b] = nd
            mx = _max2(nd)
            idx = _min2(jnp.where(nd == mx, iota2, _N))
            new_far.append(idx)
        cxyz_ref[pl.ds(i, 1), :] = jnp.concatenate(cents, axis=1)
        return jnp.concatenate(new_far, axis=1)

    lax.fori_loop(0, _M, body, jnp.zeros((1, _B), jnp.int32))


@jax.jit
def _fps(xyz4):
    return pl.pallas_call(
        _fps_body,
        out_shape=[
            jax.ShapeDtypeStruct((_M, _B), jnp.int32),
            jax.ShapeDtypeStruct((_M, 3 * _B), jnp.float32),
        ],
        scratch_shapes=[pltpu.VMEM((_B, _NR, _NC), jnp.float32)],
    )(xyz4)


_MT = 256


def _top4_body(sq_ref, xyz_ref, out_ref):
    s = sq_ref[0]
    s0 = s[:, 0:1]
    s1 = s[:, 1:2]
    s2 = s[:, 2:3]
    x0 = xyz_ref[0, 0:1, :]
    x1 = xyz_ref[0, 1:2, :]
    x2 = xyz_ref[0, 2:3, :]
    a2 = (s0 * s0 + s1 * s1) + s2 * s2
    b2 = (x0 * x0 + x1 * x1) + x2 * x2

    def _bf(v):
        return v.astype(jnp.bfloat16).astype(jnp.float32)

    ab = (_bf(s0) * _bf(x0) + _bf(s1) * _bf(x1)) + _bf(s2) * _bf(x2)
    d2 = jnp.maximum((a2 + b2) - 2.0 * ab, 0.0)
    work = jnp.sqrt(d2)
    iota = lax.broadcasted_iota(jnp.int32, (_MT, _N), 1)
    cols = []
    for k in range(_K):
        mn = jnp.min(work, axis=1, keepdims=True)
        ik = jnp.min(jnp.where(work == mn, iota, _N), axis=1, keepdims=True)
        cols.append(ik)
        if k < _K - 1:
            work = jnp.where(iota == ik, jnp.inf, work)
    out_ref[0] = jnp.concatenate(cols, axis=1)


@jax.jit
def _top4(sq0, xyz):
    return pl.pallas_call(
        _top4_body,
        grid=(_B, _M // _MT),
        in_specs=[
            pl.BlockSpec((1, _MT, 3), lambda b, m: (b, m, 0)),
            pl.BlockSpec((1, 3, _N), lambda b, m: (b, 0, 0)),
        ],
        out_specs=pl.BlockSpec((1, _MT, _K), lambda b, m: (b, m, 0)),
        out_shape=jax.ShapeDtypeStruct((_B, _M, _K), jnp.int32),
    )(sq0, xyz)


_NW = 32
_SPW = _B * _M // _NW
_CH = 4
_CS = _SPW // _CH
_CP = 128


def _sc_gather_body(xt, xyzp, gidx, outx, outc,
                    idx_v, xrows_v, crows_v, outx_v, outc_v, semx, semc):
    w = lax.axis_index("s") * 2 + lax.axis_index("c")
    for c in range(_CH):
        blk = w * _CH + c
        base = w * _SPW + c * _CS
        pltpu.sync_copy(gidx.at[blk], idx_v)
        cps = []
        for j in range(_K):
            cps.append(pltpu.async_copy(
                xt.at[idx_v.at[j]], xrows_v.at[pl.ds(j * _CS, _CS)], semx))
            cps.append(pltpu.async_copy(
                xyzp.at[idx_v.at[j]], crows_v.at[pl.ds(j * _CS, _CS)], semc))
        for cp in cps:
            cp.wait()

        def body(s, carry):
            r = _K * s
            for j in range(_D // 16):
                sl = pl.ds(j * 16, 16)
                m01 = jnp.maximum(xrows_v[r, sl], xrows_v[r + 1, sl])
                m23 = jnp.maximum(xrows_v[r + 2, sl], xrows_v[r + 3, sl])
                outx_v[s, sl] = jnp.maximum(m01, m23)
            c16 = pl.ds(0, 16)
            csum = ((crows_v[r, c16] + crows_v[r + 1, c16])
                    + crows_v[r + 2, c16]) + crows_v[r + 3, c16]
            outc_v[s, :] = csum * 0.25
            return carry

        lax.fori_loop(0, _CS, body, 0)
        pltpu.sync_copy(outx_v, outx.at[pl.ds(base, _CS)])
        pltpu.sync_copy(outc_v, outc.at[pl.ds(base, _CS)])


@jax.jit
def _sc_gather(xt, xyzp, gidx):
    from jax.experimental.pallas import tpu_sc as plsc
    mesh = plsc.VectorSubcoreMesh(core_axis_name="c", subcore_axis_name="s")
    return pl.kernel(
        _sc_gather_body,
        mesh=mesh,
        out_type=[
            jax.ShapeDtypeStruct((_B * _M, _D), jnp.float32),
            jax.ShapeDtypeStruct((_B * _M, 16), jnp.float32),
        ],
        scratch_types=[
            pltpu.VMEM((_K, _CS), jnp.int32),
            pltpu.VMEM((_K * _CS, _D), jnp.float32),
            pltpu.VMEM((_K * _CS, _CP), jnp.float32),
            pltpu.VMEM((_CS, _D), jnp.float32),
            pltpu.VMEM((_CS, 16), jnp.float32),
            pltpu.SemaphoreType.DMA,
            pltpu.SemaphoreType.DMA,
        ],
    )(xt, xyzp, gidx)


def kernel(x, xyz):
    xyz4 = xyz.reshape(_B, 3, _NR, _NC)
    ind_t, cxyz = _fps(xyz4)
    sample_ind = ind_t.T
    sq0 = cxyz.reshape(_M, _B, 3).transpose(1, 0, 2)
    neighbor_ind = _top4(sq0, xyz)

    xt = x.transpose(0, 2, 1).reshape(_B * _N, _D)
    xyzp = jnp.pad(xyz.transpose(0, 2, 1), ((0, 0), (0, 0), (0, _CP - 3)))
    xyzp = xyzp.reshape(_B * _N, _CP)
    g = neighbor_ind + (jnp.arange(_B, dtype=jnp.int32) * _N)[:, None, None]
    gidx = g.reshape(_NW * _CH, _K, _CS)
    outx, outc = _sc_gather(xt, xyzp, gidx)
    sample_x = outx.reshape(_B, _M, _D).transpose(0, 2, 1)
    sample_xyz = outc[:, :3].reshape(_B, _M, 3).transpose(0, 2, 1)
    return (sample_x, sample_xyz, sample_ind, neighbor_ind)


# speedup vs baseline: 22.5234x; 1.2020x over previous
import functools

import jax
import jax.numpy as jnp
from jax import lax
from jax.experimental import pallas as pl
from jax.experimental.pallas import tpu as pltpu

_B, _D, _N = 4, 128, 8192
_M = 2048
_K = 4
_NR, _NC = 64, 128


def _sum2(v):
    return jnp.sum(jnp.sum(v, axis=1, keepdims=True), axis=0, keepdims=True)


def _max2(v):
    return jnp.max(jnp.max(v, axis=1, keepdims=True), axis=0, keepdims=True)


def _min2(v):
    return jnp.min(jnp.min(v, axis=1, keepdims=True), axis=0, keepdims=True)


def _fps_body(xyz_ref, ind_ref, cxyz_ref, dist_ref):
    iotaf = (lax.broadcasted_iota(jnp.int32, (_NR, _NC), 0) * _NC
             + lax.broadcasted_iota(jnp.int32, (_NR, _NC), 1)
             ).astype(jnp.float32)
    for b in range(_B):
        dist_ref[b] = jnp.full((_NR, _NC), jnp.inf, jnp.float32)

    def body(i, far):
        ind_ref[pl.ds(i, 1), :] = far.astype(jnp.int32)
        cents = []
        new_far = []
        for b in range(_B):
            fb = far[0:1, b:b + 1]
            onehot = iotaf == fb
            xb = xyz_ref[b, 0]
            yb = xyz_ref[b, 1]
            zb = xyz_ref[b, 2]
            cx = _sum2(jnp.where(onehot, xb, 0.0))
            cy = _sum2(jnp.where(onehot, yb, 0.0))
            cz = _sum2(jnp.where(onehot, zb, 0.0))
            cents += [cx, cy, cz]
            dx = xb - cx
            dy = yb - cy
            dz = zb - cz
            d2 = (dx * dx + dy * dy) + dz * dz
            nd = jnp.minimum(dist_ref[b], d2)
            dist_ref[b] = nd
            mx = _max2(nd)
            idxf = _min2(jnp.where(nd == mx, iotaf, float(_N)))
            new_far.append(idxf)
        cxyz_ref[pl.ds(i, 1), :] = jnp.concatenate(cents, axis=1)
        return jnp.concatenate(new_far, axis=1)

    lax.fori_loop(0, _M, body, jnp.zeros((1, _B), jnp.float32))


@jax.jit
def _fps(xyz4):
    return pl.pallas_call(
        _fps_body,
        out_shape=[
            jax.ShapeDtypeStruct((_M, _B), jnp.int32),
            jax.ShapeDtypeStruct((_M, 3 * _B), jnp.float32),
        ],
        scratch_shapes=[pltpu.VMEM((_B, _NR, _NC), jnp.float32)],
    )(xyz4)


_MT = 256


def _top4_body(sq_ref, xyz_ref, out_ref):
    s = sq_ref[0]
    s0 = s[:, 0:1]
    s1 = s[:, 1:2]
    s2 = s[:, 2:3]
    x0 = xyz_ref[0, 0:1, :]
    x1 = xyz_ref[0, 1:2, :]
    x2 = xyz_ref[0, 2:3, :]
    a2 = (s0 * s0 + s1 * s1) + s2 * s2
    b2 = (x0 * x0 + x1 * x1) + x2 * x2

    ab = lax.dot_general(
        s.astype(jnp.bfloat16), xyz_ref[0].astype(jnp.bfloat16),
        (((1,), (0,)), ((), ())), preferred_element_type=jnp.float32)
    d2 = jnp.maximum((a2 + b2) - 2.0 * ab, 0.0)
    work = jnp.sqrt(d2)
    iota = lax.broadcasted_iota(jnp.int32, (_MT, _N), 1).astype(jnp.float32)
    cols = []
    for k in range(_K):
        mn = jnp.min(work, axis=1, keepdims=True)
        ik = jnp.min(jnp.where(work == mn, iota, float(_N)),
                     axis=1, keepdims=True)
        cols.append(ik)
        if k < _K - 1:
            work = jnp.where(iota == ik, jnp.inf, work)
    out_ref[0] = jnp.concatenate(cols, axis=1).astype(jnp.int32)


@jax.jit
def _top4(sq0, xyz):
    return pl.pallas_call(
        _top4_body,
        grid=(_B, _M // _MT),
        in_specs=[
            pl.BlockSpec((1, _MT, 3), lambda b, m: (b, m, 0)),
            pl.BlockSpec((1, 3, _N), lambda b, m: (b, 0, 0)),
        ],
        out_specs=pl.BlockSpec((1, _MT, _K), lambda b, m: (b, m, 0)),
        out_shape=jax.ShapeDtypeStruct((_B, _M, _K), jnp.int32),
    )(sq0, xyz)


_NW = 32
_SPW = _B * _M // _NW
_CH = 4
_CS = _SPW // _CH
_CP = 128


def _sc_gather_body(xt, xyzp, gidx, outx, outc,
                    idx_v, xrows_v, crows_v, outx_v, outc_v, semx, semc):
    w = lax.axis_index("s") * 2 + lax.axis_index("c")
    for c in range(_CH):
        blk = w * _CH + c
        base = w * _SPW + c * _CS
        pltpu.sync_copy(gidx.at[blk], idx_v)
        cps = []
        for j in range(_K):
            cps.append(pltpu.async_copy(
                xt.at[idx_v.at[j]], xrows_v.at[pl.ds(j * _CS, _CS)], semx))
            cps.append(pltpu.async_copy(
                xyzp.at[idx_v.at[j]], crows_v.at[pl.ds(j * _CS, _CS)], semc))
        for cp in cps:
            cp.wait()

        def body(s, carry):
            r = _K * s
            for j in range(_D // 16):
                sl = pl.ds(j * 16, 16)
                m01 = jnp.maximum(xrows_v[r, sl], xrows_v[r + 1, sl])
                m23 = jnp.maximum(xrows_v[r + 2, sl], xrows_v[r + 3, sl])
                outx_v[s, sl] = jnp.maximum(m01, m23)
            c16 = pl.ds(0, 16)
            csum = ((crows_v[r, c16] + crows_v[r + 1, c16])
                    + crows_v[r + 2, c16]) + crows_v[r + 3, c16]
            outc_v[s, :] = csum * 0.25
            return carry

        lax.fori_loop(0, _CS, body, 0)
        pltpu.sync_copy(outx_v, outx.at[pl.ds(base, _CS)])
        pltpu.sync_copy(outc_v, outc.at[pl.ds(base, _CS)])


@jax.jit
def _sc_gather(xt, xyzp, gidx):
    from jax.experimental.pallas import tpu_sc as plsc
    mesh = plsc.VectorSubcoreMesh(core_axis_name="c", subcore_axis_name="s")
    return pl.kernel(
        _sc_gather_body,
        mesh=mesh,
        out_type=[
            jax.ShapeDtypeStruct((_B * _M, _D), jnp.float32),
            jax.ShapeDtypeStruct((_B * _M, 16), jnp.float32),
        ],
        scratch_types=[
            pltpu.VMEM((_K, _CS), jnp.int32),
            pltpu.VMEM((_K * _CS, _D), jnp.float32),
            pltpu.VMEM((_K * _CS, _CP), jnp.float32),
            pltpu.VMEM((_CS, _D), jnp.float32),
            pltpu.VMEM((_CS, 16), jnp.float32),
            pltpu.SemaphoreType.DMA,
            pltpu.SemaphoreType.DMA,
        ],
    )(xt, xyzp, gidx)


def kernel(x, xyz):
    xyz4 = xyz.reshape(_B, 3, _NR, _NC)
    ind_t, cxyz = _fps(xyz4)
    sample_ind = ind_t.T
    sq0 = cxyz.reshape(_M, _B, 3).transpose(1, 0, 2)
    neighbor_ind = _top4(sq0, xyz)

    xt = x.transpose(0, 2, 1).reshape(_B * _N, _D)
    xyzp = jnp.pad(xyz.transpose(0, 2, 1), ((0, 0), (0, 0), (0, _CP - 3)))
    xyzp = xyzp.reshape(_B * _N, _CP)
    g = neighbor_ind + (jnp.arange(_B, dtype=jnp.int32) * _N)[:, None, None]
    gidx = g.reshape(_NW * _CH, _K, _CS)
    outx, outc = _sc_gather(xt, xyzp, gidx)
    sample_x = outx.reshape(_B, _M, _D).transpose(0, 2, 1)
    sample_xyz = outc[:, :3].reshape(_B, _M, 3).transpose(0, 2, 1)
    return (sample_x, sample_xyz, sample_ind, neighbor_ind)


# speedup vs baseline: 22.8122x; 1.0128x over previous
import functools

import jax
import jax.numpy as jnp
from jax import lax
from jax.experimental import pallas as pl
from jax.experimental.pallas import tpu as pltpu

_B, _D, _N = 4, 128, 8192
_M = 2048
_K = 4
_NR, _NC = 64, 128


def _sum2(v):
    return jnp.sum(jnp.sum(v, axis=1, keepdims=True), axis=0, keepdims=True)


def _max2(v):
    return jnp.max(jnp.max(v, axis=1, keepdims=True), axis=0, keepdims=True)


def _min2(v):
    return jnp.min(jnp.min(v, axis=1, keepdims=True), axis=0, keepdims=True)


def _fps_body(xyz_ref, ind_ref, cxyz_ref, dist_ref):
    iotaf = (lax.broadcasted_iota(jnp.int32, (_NR, _NC), 0) * _NC
             + lax.broadcasted_iota(jnp.int32, (_NR, _NC), 1)
             ).astype(jnp.float32)
    for b in range(_B):
        dist_ref[b] = jnp.full((_NR, _NC), jnp.inf, jnp.float32)

    def step(far):
        cents = []
        new_far = []
        for b in range(_B):
            fb = far[0:1, b:b + 1]
            onehot = iotaf == fb
            xb = xyz_ref[b, 0]
            yb = xyz_ref[b, 1]
            zb = xyz_ref[b, 2]
            cx = _sum2(jnp.where(onehot, xb, 0.0))
            cy = _sum2(jnp.where(onehot, yb, 0.0))
            cz = _sum2(jnp.where(onehot, zb, 0.0))
            cents += [cx, cy, cz]
            dx = xb - cx
            dy = yb - cy
            dz = zb - cz
            d2 = (dx * dx + dy * dy) + dz * dz
            nd = jnp.minimum(dist_ref[b], d2)
            dist_ref[b] = nd
            mx = _max2(nd)
            idxf = _min2(jnp.where(nd == mx, iotaf, float(_N)))
            new_far.append(idxf)
        return (jnp.concatenate(new_far, axis=1),
                jnp.concatenate(cents, axis=1))

    def body(i, far):
        far1, cent0 = step(far)
        far2, cent1 = step(far1)
        ind_ref[pl.ds(2 * i, 2), :] = jnp.concatenate(
            [far, far1], axis=0).astype(jnp.int32)
        cxyz_ref[pl.ds(2 * i, 2), :] = jnp.concatenate(
            [cent0, cent1], axis=0)
        return far2

    lax.fori_loop(0, _M // 2, body, jnp.zeros((1, _B), jnp.float32))


@jax.jit
def _fps(xyz4):
    return pl.pallas_call(
        _fps_body,
        out_shape=[
            jax.ShapeDtypeStruct((_M, _B), jnp.int32),
            jax.ShapeDtypeStruct((_M, 3 * _B), jnp.float32),
        ],
        scratch_shapes=[pltpu.VMEM((_B, _NR, _NC), jnp.float32)],
    )(xyz4)


_MT = 256


def _top4_body(sq_ref, xyz_ref, out_ref):
    s = sq_ref[0]
    s0 = s[:, 0:1]
    s1 = s[:, 1:2]
    s2 = s[:, 2:3]
    x0 = xyz_ref[0, 0:1, :]
    x1 = xyz_ref[0, 1:2, :]
    x2 = xyz_ref[0, 2:3, :]
    a2 = (s0 * s0 + s1 * s1) + s2 * s2
    b2 = (x0 * x0 + x1 * x1) + x2 * x2

    ab = lax.dot_general(
        s.astype(jnp.bfloat16), xyz_ref[0].astype(jnp.bfloat16),
        (((1,), (0,)), ((), ())), preferred_element_type=jnp.float32)
    d2 = jnp.maximum((a2 + b2) - 2.0 * ab, 0.0)
    work = jnp.sqrt(d2)
    iota = lax.broadcasted_iota(jnp.int32, (_MT, _N), 1).astype(jnp.float32)
    cols = []
    for k in range(_K):
        mn = jnp.min(work, axis=1, keepdims=True)
        ik = jnp.min(jnp.where(work == mn, iota, float(_N)),
                     axis=1, keepdims=True)
        cols.append(ik)
        if k < _K - 1:
            work = jnp.where(iota == ik, jnp.inf, work)
    out_ref[0] = jnp.concatenate(cols, axis=1).astype(jnp.int32)


@jax.jit
def _top4(sq0, xyz):
    return pl.pallas_call(
        _top4_body,
        grid=(_B, _M // _MT),
        in_specs=[
            pl.BlockSpec((1, _MT, 3), lambda b, m: (b, m, 0)),
            pl.BlockSpec((1, 3, _N), lambda b, m: (b, 0, 0)),
        ],
        out_specs=pl.BlockSpec((1, _MT, _K), lambda b, m: (b, m, 0)),
        out_shape=jax.ShapeDtypeStruct((_B, _M, _K), jnp.int32),
    )(sq0, xyz)


_NW = 32
_SPW = _B * _M // _NW
_CH = 4
_CS = _SPW // _CH
_CP = 128


def _sc_gather_body(xt, xyzp, gidx, outx, outc,
                    idx_v, xrows_v, crows_v, outx_v, outc_v, semx, semc):
    w = lax.axis_index("s") * 2 + lax.axis_index("c")
    for c in range(_CH):
        blk = w * _CH + c
        base = w * _SPW + c * _CS
        pltpu.sync_copy(gidx.at[blk], idx_v)
        cps = []
        for j in range(_K):
            cps.append(pltpu.async_copy(
                xt.at[idx_v.at[j]], xrows_v.at[pl.ds(j * _CS, _CS)], semx))
            cps.append(pltpu.async_copy(
                xyzp.at[idx_v.at[j]], crows_v.at[pl.ds(j * _CS, _CS)], semc))
        for cp in cps:
            cp.wait()

        def body(s, carry):
            r = _K * s
            for j in range(_D // 16):
                sl = pl.ds(j * 16, 16)
                m01 = jnp.maximum(xrows_v[r, sl], xrows_v[r + 1, sl])
                m23 = jnp.maximum(xrows_v[r + 2, sl], xrows_v[r + 3, sl])
                outx_v[s, sl] = jnp.maximum(m01, m23)
            c16 = pl.ds(0, 16)
            csum = ((crows_v[r, c16] + crows_v[r + 1, c16])
                    + crows_v[r + 2, c16]) + crows_v[r + 3, c16]
            outc_v[s, :] = csum * 0.25
            return carry

        lax.fori_loop(0, _CS, body, 0)
        pltpu.sync_copy(outx_v, outx.at[pl.ds(base, _CS)])
        pltpu.sync_copy(outc_v, outc.at[pl.ds(base, _CS)])


@jax.jit
def _sc_gather(xt, xyzp, gidx):
    from jax.experimental.pallas import tpu_sc as plsc
    mesh = plsc.VectorSubcoreMesh(core_axis_name="c", subcore_axis_name="s")
    return pl.kernel(
        _sc_gather_body,
        mesh=mesh,
        out_type=[
            jax.ShapeDtypeStruct((_B * _M, _D), jnp.float32),
            jax.ShapeDtypeStruct((_B * _M, 16), jnp.float32),
        ],
        scratch_types=[
            pltpu.VMEM((_K, _CS), jnp.int32),
            pltpu.VMEM((_K * _CS, _D), jnp.float32),
            pltpu.VMEM((_K * _CS, _CP), jnp.float32),
            pltpu.VMEM((_CS, _D), jnp.float32),
            pltpu.VMEM((_CS, 16), jnp.float32),
            pltpu.SemaphoreType.DMA,
            pltpu.SemaphoreType.DMA,
        ],
    )(xt, xyzp, gidx)


def kernel(x, xyz):
    xyz4 = xyz.reshape(_B, 3, _NR, _NC)
    ind_t, cxyz = _fps(xyz4)
    sample_ind = ind_t.T
    sq0 = cxyz.reshape(_M, _B, 3).transpose(1, 0, 2)
    neighbor_ind = _top4(sq0, xyz)

    xt = x.transpose(0, 2, 1).reshape(_B * _N, _D)
    xyzp = jnp.pad(xyz.transpose(0, 2, 1), ((0, 0), (0, 0), (0, _CP - 3)))
    xyzp = xyzp.reshape(_B * _N, _CP)
    g = neighbor_ind + (jnp.arange(_B, dtype=jnp.int32) * _N)[:, None, None]
    gidx = g.reshape(_NW * _CH, _K, _CS)
    outx, outc = _sc_gather(xt, xyzp, gidx)
    sample_x = outx.reshape(_B, _M, _D).transpose(0, 2, 1)
    sample_xyz = outc[:, :3].reshape(_B, _M, 3).transpose(0, 2, 1)
    return (sample_x, sample_xyz, sample_ind, neighbor_ind)
